# baseline (speedup 1.0000x reference)
import functools
import math

import jax
import jax.numpy as jnp
from jax import lax
from jax.experimental import pallas as pl
from jax.experimental.pallas import tpu as pltpu
from jax.experimental.pallas import tpu_sc as plsc

_DX = 1.0 / 64.0
_WEIGHT = 10.0
_E_IN = 2.0
_GCONST = 1.0 / (_E_IN * 4.0 * math.pi)

_LANE = 128
_NC = 2
_NS = 16
_NW = _NC * _NS


def _sc_gather(flat_b, idx4d):
    _, parts, rows_p, _ = idx4d.shape
    assert 2 * parts == _NW
    mesh = plsc.VectorSubcoreMesh(core_axis_name="c", subcore_axis_name="s")

    @functools.partial(
        pl.kernel,
        mesh=mesh,
        out_type=jax.ShapeDtypeStruct(idx4d.shape, jnp.float32),
        scratch_types=[
            pltpu.VMEM((rows_p, _LANE), jnp.int32),
            pltpu.VMEM((rows_p, _LANE), jnp.float32),
            pltpu.SemaphoreType.DMA,
        ],
    )
    def gather_kernel(flat_hbm, idx_hbm, out_hbm, idx_v, val_v, sem):
        wid = lax.axis_index("s") * _NC + lax.axis_index("c")
        kind = wid // parts
        part = wid % parts
        pltpu.sync_copy(idx_hbm.at[kind].at[part], idx_v)
        copies = [
            pltpu.async_copy(flat_hbm.at[idx_v.at[k]], val_v.at[k], sem)
            for k in range(rows_p)
        ]
        for c in copies:
            c.wait()
        pltpu.sync_copy(val_v, out_hbm.at[kind].at[part])

    return gather_kernel(flat_b, idx4d)


def _tc_gfield(idx3d, q, xqx, xqy, xqz):
    nq = q.shape[0]
    shape3d = idx3d.shape
    unroll = 4

    def body(idx_ref, q_ref, xqx_ref, xqy_ref, xqz_ref, g_ref):
        idx = idx_ref[...]
        px = (idx >> 12).astype(jnp.float32) * _DX
        py = ((idx >> 6) & 63).astype(jnp.float32) * _DX
        pz = (idx & 63).astype(jnp.float32) * _DX

        def charge(jj, acc):
            j0 = jj * unroll
            for u in range(unroll):
                j = j0 + u
                dx = px - xqx_ref[j]
                dy = py - xqy_ref[j]
                dz = pz - xqz_ref[j]
                r2 = dx * dx + dy * dy + dz * dz
                acc = acc + q_ref[j] * lax.rsqrt(r2)
            return acc

        g = lax.fori_loop(0, nq // unroll, charge,
                          jnp.zeros(shape3d, jnp.float32))
        g_ref[...] = g * jnp.float32(_GCONST)

    return pl.pallas_call(
        body,
        out_shape=jax.ShapeDtypeStruct(shape3d, jnp.float32),
        in_specs=[
            pl.BlockSpec(memory_space=pltpu.VMEM),
            pl.BlockSpec(memory_space=pltpu.SMEM),
            pl.BlockSpec(memory_space=pltpu.SMEM),
            pl.BlockSpec(memory_space=pltpu.SMEM),
            pl.BlockSpec(memory_space=pltpu.SMEM),
        ],
        out_specs=pl.BlockSpec(memory_space=pltpu.VMEM),
    )(idx3d, q, xqx, xqy, xqz)


def _tc_combine(vals_list, g3d, n):
    parts, rows_p, _ = g3d.shape
    slab = rows_p * _LANE
    nb = len(vals_list)

    def body(*refs):
        vals_refs = refs[:nb]
        g_ref = refs[nb]
        out_ref = refs[nb + 1]
        shape2d = (rows_p, _LANE)
        rows = lax.broadcasted_iota(jnp.int32, shape2d, 0)
        cols = lax.broadcasted_iota(jnp.int32, shape2d, 1)
        pos = rows * _LANE + cols

        total = jnp.float32(0.0)
        for p in range(parts):
            g = g_ref[p]
            valid = pos + p * slab < n
            for b in range(nb):
                t = vals_refs[b][0, p] + g - vals_refs[b][1, p]
                t = jnp.where(valid, t, jnp.float32(0.0))
                total = total + jnp.sum(t * t)
        out_ref[0, 0] = total * jnp.float32(_WEIGHT / (n * nb))

    out = pl.pallas_call(
        body,
        out_shape=jax.ShapeDtypeStruct((1, 1), jnp.float32),
        in_specs=[pl.BlockSpec(memory_space=pltpu.VMEM)] * (nb + 1),
        out_specs=pl.BlockSpec(memory_space=pltpu.SMEM),
    )(*vals_list, g3d)
    return out[0, 0]


def kernel(output, q, xq, points, inner_idx, outer_idx, x_idx, y_idx, z_idx,
           normal_x, normal_y, normal_z):
    nb = output.shape[0]
    ngrid = output.shape[2] * output.shape[3] * output.shape[4]
    n = inner_idx.shape[0]

    parts = _NW // 2
    seg = -(-n // (parts * _LANE)) * (parts * _LANE)
    rows_p = seg // (parts * _LANE)
    pad = seg - n

    zpad = jnp.zeros((pad,), jnp.int32)
    idx_in = jnp.concatenate([inner_idx, zpad]).reshape(parts, rows_p, _LANE)
    idx_out = jnp.concatenate([outer_idx, zpad]).reshape(parts, rows_p, _LANE)
    idx4d = jnp.stack([idx_in, idx_out])

    vals = [_sc_gather(output[b, 0].reshape(ngrid), idx4d)
            for b in range(nb)]
    g3d = _tc_gfield(idx_in, q, xq[:, 0], xq[:, 1], xq[:, 2])

    return _tc_combine(vals, g3d, n)


# speedup vs baseline: 1.6288x; 1.6288x over previous
import functools
import math

import jax
import jax.numpy as jnp
from jax import lax
from jax.experimental import pallas as pl
from jax.experimental.pallas import tpu as pltpu
from jax.experimental.pallas import tpu_sc as plsc

_DX = 1.0 / 64.0
_WEIGHT = 10.0
_E_IN = 2.0
_GCONST = 1.0 / (_E_IN * 4.0 * math.pi)

_LANE = 128
_NC = 2
_NS = 16
_NW = _NC * _NS


def _sc_gather(flat1d, idx5d):
    _, nb, parts, rows_p, _ = idx5d.shape
    assert 2 * nb * parts == _NW
    mesh = plsc.VectorSubcoreMesh(core_axis_name="c", subcore_axis_name="s")

    @functools.partial(
        pl.kernel,
        mesh=mesh,
        out_type=jax.ShapeDtypeStruct(idx5d.shape, jnp.float32),
        scratch_types=[
            pltpu.VMEM((rows_p, _LANE), jnp.int32),
            pltpu.VMEM((rows_p, _LANE), jnp.float32),
            pltpu.SemaphoreType.DMA,
        ],
    )
    def gather_kernel(flat_hbm, idx_hbm, out_hbm, idx_v, val_v, sem):
        wid = lax.axis_index("s") * _NC + lax.axis_index("c")
        kind = wid // (nb * parts)
        b = (wid // parts) % nb
        part = wid % parts
        pltpu.sync_copy(idx_hbm.at[kind].at[b].at[part], idx_v)
        copies = [
            pltpu.async_copy(flat_hbm.at[idx_v.at[k]], val_v.at[k], sem)
            for k in range(rows_p)
        ]
        for c in copies:
            c.wait()
        pltpu.sync_copy(val_v, out_hbm.at[kind].at[b].at[part])

    return gather_kernel(flat1d, idx5d)


def _tc_gfield(idx3d, q, xqx, xqy, xqz):
    nq = q.shape[0]
    shape3d = idx3d.shape
    unroll = 4

    def body(idx_ref, q_ref, xqx_ref, xqy_ref, xqz_ref, g_ref):
        idx = idx_ref[...]
        px = (idx >> 12).astype(jnp.float32) * _DX
        py = ((idx >> 6) & 63).astype(jnp.float32) * _DX
        pz = (idx & 63).astype(jnp.float32) * _DX

        def charge(jj, acc):
            j0 = jj * unroll
            for u in range(unroll):
                j = j0 + u
                dx = px - xqx_ref[j]
                dy = py - xqy_ref[j]
                dz = pz - xqz_ref[j]
                r2 = dx * dx + dy * dy + dz * dz
                acc = acc + q_ref[j] * lax.rsqrt(r2)
            return acc

        g = lax.fori_loop(0, nq // unroll, charge,
                          jnp.zeros(shape3d, jnp.float32))
        g_ref[...] = g * jnp.float32(_GCONST)

    return pl.pallas_call(
        body,
        out_shape=jax.ShapeDtypeStruct(shape3d, jnp.float32),
        in_specs=[
            pl.BlockSpec(memory_space=pltpu.VMEM),
            pl.BlockSpec(memory_space=pltpu.SMEM),
            pl.BlockSpec(memory_space=pltpu.SMEM),
            pl.BlockSpec(memory_space=pltpu.SMEM),
            pl.BlockSpec(memory_space=pltpu.SMEM),
        ],
        out_specs=pl.BlockSpec(memory_space=pltpu.VMEM),
    )(idx3d, q, xqx, xqy, xqz)


def _tc_combine(vals, g3d, n, nb):
    parts, rows_p, _ = g3d.shape
    slab = rows_p * _LANE

    def body(vals_ref, g_ref, out_ref):
        shape2d = (rows_p, _LANE)
        rows = lax.broadcasted_iota(jnp.int32, shape2d, 0)
        cols = lax.broadcasted_iota(jnp.int32, shape2d, 1)
        pos = rows * _LANE + cols

        total = jnp.float32(0.0)
        for p in range(parts):
            g = g_ref[p]
            valid = pos + p * slab < n
            for b in range(nb):
                t = vals_ref[0, b, p] + g - vals_ref[1, b, p]
                t = jnp.where(valid, t, jnp.float32(0.0))
                total = total + jnp.sum(t * t)
        out_ref[0, 0] = total * jnp.float32(_WEIGHT / (n * nb))

    out = pl.pallas_call(
        body,
        out_shape=jax.ShapeDtypeStruct((1, 1), jnp.float32),
        in_specs=[
            pl.BlockSpec(memory_space=pltpu.VMEM),
            pl.BlockSpec(memory_space=pltpu.VMEM),
        ],
        out_specs=pl.BlockSpec(memory_space=pltpu.SMEM),
    )(vals, g3d)
    return out[0, 0]


def kernel(output, q, xq, points, inner_idx, outer_idx, x_idx, y_idx, z_idx,
           normal_x, normal_y, normal_z):
    nb = output.shape[0]
    ngrid = output.shape[2] * output.shape[3] * output.shape[4]
    n = inner_idx.shape[0]

    parts = _NW // (2 * nb)
    seg = -(-n // (parts * _LANE)) * (parts * _LANE)
    rows_p = seg // (parts * _LANE)
    pad = seg - n

    zpad = jnp.zeros((pad,), jnp.int32)
    idx_in = jnp.concatenate([inner_idx, zpad]).reshape(parts, rows_p, _LANE)
    idx_out = jnp.concatenate([outer_idx, zpad]).reshape(parts, rows_p, _LANE)
    offs = (jnp.arange(nb, dtype=jnp.int32) * ngrid).reshape(nb, 1, 1, 1)
    idx5d = jnp.stack([idx_in, idx_out])[:, None] + offs[None]

    flat1d = output.reshape(nb * ngrid)
    gathered = _sc_gather(flat1d, idx5d)
    g3d = _tc_gfield(idx_in, q, xq[:, 0], xq[:, 1], xq[:, 2])

    return _tc_combine(gathered, g3d, n, nb)
